# baseline (speedup 1.0000x reference)
import jax
import jax.numpy as jnp
from jax import lax
from jax.experimental import pallas as pl
from jax.experimental.pallas import tpu as pltpu
from jax.experimental.pallas import tpu_sc as plsc

N = 16384
K = 1024
D = 64

NH = 1
H = N // NH
BN = 4096
SUB = 1024
NSUB = BN // SUB
NBH = H // BN


def _assign_body(xt_ref, c_ref, assign_ref, sumd_ref):
    i = pl.program_id(0)
    c = c_ref[...]
    c2 = jnp.sum(c * c, axis=1, keepdims=True)

    @pl.when(i == 0)
    def _():
        sumd_ref[...] = jnp.zeros((1, 1), jnp.float32)

    sub_sum = jnp.zeros((1, 1), jnp.float32)
    for s in range(NSUB):
        xt = jnp.transpose(xt_ref[pl.ds(s * SUB, SUB), :])
        x2 = jnp.sum(xt * xt, axis=0, keepdims=True)
        prod = lax.dot_general(c, xt, (((1,), (0,)), ((), ())),
                               preferred_element_type=jnp.float32)
        dist = (x2 - 2.0 * prod) + c2
        minv = jnp.min(dist, axis=0, keepdims=True)
        kidx = lax.broadcasted_iota(jnp.int32, (K, SUB), 0)
        cand = jnp.where(dist == minv, kidx, jnp.int32(K))
        assign_ref[0, 0, pl.ds(s * SUB, SUB)] = jnp.min(cand, axis=0)
        sub_sum = sub_sum + jnp.sum(minv).reshape(1, 1)

    sumd_ref[...] += sub_sum


def _assign_stage(xt, centroids, h):
    assign3, sumd = pl.pallas_call(
        _assign_body,
        grid=(NBH,),
        in_specs=[
            pl.BlockSpec((BN, D), lambda i: (i + h * NBH, 0)),
            pl.BlockSpec((K, D), lambda i: (0, 0)),
        ],
        out_specs=[
            pl.BlockSpec((1, 1, BN), lambda i: (i, 0, 0)),
            pl.BlockSpec((1, 1), lambda i: (0, 0)),
        ],
        out_shape=[
            jax.ShapeDtypeStruct((NBH, 1, BN), jnp.int32),
            jax.ShapeDtypeStruct((1, 1), jnp.float32),
        ],
    )(xt, centroids)
    return assign3.reshape(H), sumd


CHUNK = 128
NWORK = 32
PTS_PER_WORKER = H // NWORK
NCHUNKS = PTS_PER_WORKER // CHUNK
ROWS_PER_TILE = K // 16


def _make_sc_body(h):
    def _sc_body(assign_hbm, x_hbm, psums_hbm, pcnts_hbm,
                 idx_v, x_v, ones_v, row_v, cnt_v, sums_sh, cnts_sh,
                 lsem0, lsem1, ssem):
        cid = lax.axis_index("c")
        sid = lax.axis_index("s")
        wid = cid * 16 + sid
        zero16 = jnp.zeros((16,), jnp.float32)
        one16 = jnp.ones((16,), jnp.float32)

        ld_idx = pltpu.async_copy(assign_hbm.at[pl.ds(wid * NCHUNKS, NCHUNKS)],
                                  idx_v, lsem0)
        ld_x = pltpu.async_copy(
            x_hbm.at[pl.ds(h * H + wid * PTS_PER_WORKER, PTS_PER_WORKER)],
            x_v, lsem1)

        def fill_const(r, _):
            for cc in range(D // 16):
                row_v[r, pl.ds(cc * 16, 16)] = zero16
            cnt_v[r, :] = zero16
            return 0

        lax.fori_loop(0, ROWS_PER_TILE, fill_const, 0)

        def fill_ones(r, _):
            ones_v[r, :] = one16
            return 0

        lax.fori_loop(0, CHUNK, fill_ones, 0)

        rbase = sid * ROWS_PER_TILE
        pltpu.sync_copy(row_v, sums_sh.at[pl.ds(rbase, ROWS_PER_TILE)])
        pltpu.sync_copy(cnt_v, cnts_sh.at[pl.ds(rbase, ROWS_PER_TILE)])
        plsc.subcore_barrier()
        ld_idx.wait()
        ld_x.wait()

        descs = []
        for j in range(NCHUNKS):
            descs.append(pltpu.async_copy(
                x_v.at[pl.ds(j * CHUNK, CHUNK)], sums_sh.at[idx_v.at[j]],
                ssem, add=True))
            descs.append(pltpu.async_copy(
                ones_v, cnts_sh.at[idx_v.at[j]], ssem, add=True))
        for dsc in descs:
            dsc.wait()
        plsc.subcore_barrier()

        pltpu.sync_copy(sums_sh.at[pl.ds(rbase, ROWS_PER_TILE)], row_v)
        pltpu.sync_copy(cnts_sh.at[pl.ds(rbase, ROWS_PER_TILE)], cnt_v)
        pltpu.sync_copy(row_v, psums_hbm.at[cid, pl.ds(rbase, ROWS_PER_TILE)])
        pltpu.sync_copy(cnt_v, pcnts_hbm.at[cid, pl.ds(rbase, ROWS_PER_TILE)])

    return _sc_body


def _update_stage(assignments_h, x, h):
    assign2d = assignments_h.reshape(H // CHUNK, CHUNK)
    mesh = plsc.VectorSubcoreMesh(core_axis_name="c", subcore_axis_name="s")
    return pl.kernel(
        _make_sc_body(h),
        out_type=[
            jax.ShapeDtypeStruct((2, K, D), jnp.float32),
            jax.ShapeDtypeStruct((2, K, 16), jnp.float32),
        ],
        mesh=mesh,
        scratch_types=[
            pltpu.VMEM((NCHUNKS, CHUNK), jnp.int32),
            pltpu.VMEM((PTS_PER_WORKER, D), jnp.float32),
            pltpu.VMEM((CHUNK, 16), jnp.float32),
            pltpu.VMEM((ROWS_PER_TILE, D), jnp.float32),
            pltpu.VMEM((ROWS_PER_TILE, 16), jnp.float32),
            pltpu.VMEM_SHARED((K, D), jnp.float32),
            pltpu.VMEM_SHARED((K, 16), jnp.float32),
            pltpu.SemaphoreType.DMA,
            pltpu.SemaphoreType.DMA,
            pltpu.SemaphoreType.DMA,
        ],
    )(assign2d, x)


def _combine_body(ps_ref, pc_ref, out_ref):
    s = ps_ref[0] + ps_ref[1]
    cnt = pc_ref[0, :, 0:1] + pc_ref[1, :, 0:1]
    out_ref[...] = s / cnt


def _combine_stage(psums, pcnts):
    return pl.pallas_call(
        _combine_body,
        out_shape=jax.ShapeDtypeStruct((K, D), jnp.float32),
    )(psums, pcnts)


def kernel(x, centroids):
    assignments, sumd = _assign_stage(x, centroids, 0)
    psums, pcnts = _update_stage(assignments, x, 0)
    updated_centroids = _combine_stage(psums, pcnts)
    return assignments, updated_centroids, sumd[0, 0]


# speedup vs baseline: 1.0044x; 1.0044x over previous
import jax
import jax.numpy as jnp
from jax import lax
from jax.experimental import pallas as pl
from jax.experimental.pallas import tpu as pltpu
from jax.experimental.pallas import tpu_sc as plsc

N = 16384
K = 1024
D = 64

NH = 1
H = N // NH
BN = 4096
SUB = 1024
NSUB = BN // SUB
NBH = H // BN


def _assign_body(xt_ref, c_ref, assign_ref, sumd_ref):
    i = pl.program_id(0)
    c = c_ref[...]
    c2 = jnp.sum(c * c, axis=1, keepdims=True)

    @pl.when(i == 0)
    def _():
        sumd_ref[...] = jnp.zeros((1, 1), jnp.float32)

    sub_sum = jnp.zeros((1, 1), jnp.float32)
    for s in range(NSUB):
        xt = xt_ref[:, pl.ds(s * SUB, SUB)]
        x2 = jnp.sum(xt * xt, axis=0, keepdims=True)
        prod = lax.dot_general(c, xt, (((1,), (0,)), ((), ())),
                               preferred_element_type=jnp.float32)
        dist = (x2 - 2.0 * prod) + c2
        minv = jnp.min(dist, axis=0, keepdims=True)
        kidx = lax.broadcasted_iota(jnp.int32, (K, SUB), 0)
        cand = jnp.where(dist == minv, kidx, jnp.int32(K))
        assign_ref[pl.ds(s * SUB, SUB)] = jnp.min(cand, axis=0)
        sub_sum = sub_sum + jnp.sum(minv).reshape(1, 1)

    sumd_ref[...] += sub_sum


def _assign_stage(xt, centroids, h):
    assign3, sumd = pl.pallas_call(
        _assign_body,
        grid=(NBH,),
        in_specs=[
            pl.BlockSpec((D, BN), lambda i: (0, i + h * NBH)),
            pl.BlockSpec((K, D), lambda i: (0, 0)),
        ],
        out_specs=[
            pl.BlockSpec((BN,), lambda i: (i,)),
            pl.BlockSpec((1, 1), lambda i: (0, 0)),
        ],
        out_shape=[
            jax.ShapeDtypeStruct((H,), jnp.int32),
            jax.ShapeDtypeStruct((1, 1), jnp.float32),
        ],
    )(xt, centroids)
    return assign3, sumd


CHUNK = 128
NWORK = 32
PTS_PER_WORKER = H // NWORK
NCHUNKS = PTS_PER_WORKER // CHUNK
ROWS_PER_TILE = K // 16


def _make_sc_body(h):
    def _sc_body(assign_hbm, x_hbm, psums_hbm, pcnts_hbm,
                 idx_v, x_v, ones_v, row_v, cnt_v, sums_sh, cnts_sh,
                 lsem0, lsem1, ssem):
        cid = lax.axis_index("c")
        sid = lax.axis_index("s")
        wid = cid * 16 + sid
        zero16 = jnp.zeros((16,), jnp.float32)
        one16 = jnp.ones((16,), jnp.float32)

        ld_idx = pltpu.async_copy(assign_hbm.at[pl.ds(wid * NCHUNKS, NCHUNKS)],
                                  idx_v, lsem0)
        ld_x = pltpu.async_copy(
            x_hbm.at[pl.ds(h * H + wid * PTS_PER_WORKER, PTS_PER_WORKER)],
            x_v, lsem1)

        def fill_const(r, _):
            for cc in range(D // 16):
                row_v[r, pl.ds(cc * 16, 16)] = zero16
            cnt_v[r, :] = zero16
            return 0

        lax.fori_loop(0, ROWS_PER_TILE, fill_const, 0)

        def fill_ones(r, _):
            ones_v[r, :] = one16
            return 0

        lax.fori_loop(0, CHUNK, fill_ones, 0)

        rbase = sid * ROWS_PER_TILE
        pltpu.sync_copy(row_v, sums_sh.at[pl.ds(rbase, ROWS_PER_TILE)])
        pltpu.sync_copy(cnt_v, cnts_sh.at[pl.ds(rbase, ROWS_PER_TILE)])
        plsc.subcore_barrier()
        ld_idx.wait()
        ld_x.wait()

        descs = []
        for j in range(NCHUNKS):
            descs.append(pltpu.async_copy(
                x_v.at[pl.ds(j * CHUNK, CHUNK)], sums_sh.at[idx_v.at[j]],
                ssem, add=True))
            descs.append(pltpu.async_copy(
                ones_v, cnts_sh.at[idx_v.at[j]], ssem, add=True))
        for dsc in descs:
            dsc.wait()
        plsc.subcore_barrier()

        pltpu.sync_copy(sums_sh.at[pl.ds(rbase, ROWS_PER_TILE)], row_v)
        pltpu.sync_copy(cnts_sh.at[pl.ds(rbase, ROWS_PER_TILE)], cnt_v)
        pltpu.sync_copy(row_v, psums_hbm.at[cid, pl.ds(rbase, ROWS_PER_TILE)])
        pltpu.sync_copy(cnt_v, pcnts_hbm.at[cid, pl.ds(rbase, ROWS_PER_TILE)])

    return _sc_body


def _update_stage(assignments_h, x, h):
    assign2d = assignments_h.reshape(H // CHUNK, CHUNK)
    mesh = plsc.VectorSubcoreMesh(core_axis_name="c", subcore_axis_name="s")
    return pl.kernel(
        _make_sc_body(h),
        out_type=[
            jax.ShapeDtypeStruct((2, K, D), jnp.float32),
            jax.ShapeDtypeStruct((2, K, 16), jnp.float32),
        ],
        mesh=mesh,
        scratch_types=[
            pltpu.VMEM((NCHUNKS, CHUNK), jnp.int32),
            pltpu.VMEM((PTS_PER_WORKER, D), jnp.float32),
            pltpu.VMEM((CHUNK, 16), jnp.float32),
            pltpu.VMEM((ROWS_PER_TILE, D), jnp.float32),
            pltpu.VMEM((ROWS_PER_TILE, 16), jnp.float32),
            pltpu.VMEM_SHARED((K, D), jnp.float32),
            pltpu.VMEM_SHARED((K, 16), jnp.float32),
            pltpu.SemaphoreType.DMA,
            pltpu.SemaphoreType.DMA,
            pltpu.SemaphoreType.DMA,
        ],
    )(assign2d, x)


def _combine_body(ps_ref, pc_ref, out_ref):
    s = ps_ref[0] + ps_ref[1]
    cnt = pc_ref[0, :, 0:1] + pc_ref[1, :, 0:1]
    out_ref[...] = s / cnt


def _combine_stage(psums, pcnts):
    return pl.pallas_call(
        _combine_body,
        out_shape=jax.ShapeDtypeStruct((K, D), jnp.float32),
    )(psums, pcnts)


def kernel(x, centroids):
    xt = jnp.swapaxes(x, 0, 1)
    assignments, sumd = _assign_stage(xt, centroids, 0)
    psums, pcnts = _update_stage(assignments, x, 0)
    updated_centroids = _combine_stage(psums, pcnts)
    return assignments, updated_centroids, sumd[0, 0]
